# Initial kernel scaffold; baseline (speedup 1.0000x reference)
import jax
import jax.numpy as jnp
from jax.experimental import pallas as pl


def kernel(z_e, codebook_pca, W, b, token_id):
    raise NotImplementedError("write your pallas kernel here")


# speedup vs baseline: 2.0993x; 2.0993x over previous
import functools

import jax
import jax.numpy as jnp
from jax import lax
from jax.experimental import pallas as pl
from jax.experimental.pallas import tpu as pltpu
from jax.experimental.pallas import tpu_sc as plsc

N, D, K, LLM = 4096, 256, 8192, 4096
WORD = 8
B_GROUPS = N // WORD
KT = 512
K_TILES = K // KT
BIG = 0x7FFFFFFF

NW = 32
BPW = N // NW


NB = 512
N_TILES = N // NB


def _topk_body(cb_ref, w_ref, b_ref, z_ref, mapped_ref, t8v_out, t8i_out,
               m_ref, msq_ref, t8v_ref, t8i_ref):
    k = pl.program_id(0)
    j = pl.program_id(1)

    @pl.when(k == 0)
    def _init():
        sl = pl.ds(j * NB, NB)
        t8v_ref[sl, :] = jnp.full((NB, WORD), jnp.inf, jnp.float32)
        t8i_ref[sl, :] = jnp.zeros((NB, WORD), jnp.int32)

    @pl.when(j == 0)
    def _mapped():
        m = lax.dot_general(cb_ref[...], w_ref[...], (((1,), (1,)), ((), ())),
                            preferred_element_type=jnp.float32) + b_ref[...]
        mapped_ref[...] = m
        m_ref[...] = m
        msq_ref[...] = jnp.sum(m * m, axis=1)[None, :]

    z = z_ref[...]
    zsq = jnp.sum(z * z, axis=1, keepdims=True)
    m = m_ref[...]
    dot = lax.dot_general(z, m, (((1,), (1,)), ((), ())),
                          preferred_element_type=jnp.float32)
    dist = (zsq + msq_ref[...]) - 2.0 * dot

    sl = pl.ds(j * NB, NB)
    vals = jnp.concatenate([t8v_ref[sl, :], dist], axis=1)
    tile_idx = k * KT + lax.broadcasted_iota(jnp.int32, (NB, KT), 1)
    glob = jnp.concatenate([t8i_ref[sl, :], tile_idx], axis=1)
    nv, ni = [], []
    for _ in range(WORD):
        mn = jnp.min(vals, axis=1, keepdims=True)
        eq = vals == mn
        ch = jnp.min(jnp.where(eq, glob, BIG), axis=1, keepdims=True)
        nv.append(mn)
        ni.append(ch)
        vals = jnp.where(eq & (glob == ch), jnp.inf, vals)
    newv = jnp.concatenate(nv, axis=1)
    newi = jnp.concatenate(ni, axis=1)
    t8v_ref[sl, :] = newv
    t8i_ref[sl, :] = newi
    t8v_out[...] = newv
    t8i_out[...] = newi


def _greedy_body(v_ref, i_ref, out_ref):
    v = v_ref[...]
    ix = i_ref[...]
    chosen = []
    for j in range(WORD):
        vj = v[:, j * WORD:(j + 1) * WORD]
        ij = ix[:, j * WORD:(j + 1) * WORD]
        eff = vj
        for ch_prev in chosen:
            eff = jnp.where(ij == ch_prev, jnp.inf, eff)
        mn = jnp.min(eff, axis=1, keepdims=True)
        ch = jnp.min(jnp.where(eff == mn, ij, BIG), axis=1, keepdims=True)
        chosen.append(ch)
    out_ref[...] = jnp.concatenate(chosen, axis=1)


@functools.cache
def _build_sc_gather():
    mesh = plsc.VectorSubcoreMesh(core_axis_name="c", subcore_axis_name="s")

    @functools.partial(
        pl.kernel,
        mesh=mesh,
        out_type=[
            jax.ShapeDtypeStruct((N, D), jnp.float32),
            jax.ShapeDtypeStruct((N,), jnp.int32),
            jax.ShapeDtypeStruct((NW, 16), jnp.float32),
        ],
        scratch_types=[
            pltpu.VMEM((BPW,), jnp.int32),
            pltpu.VMEM((BPW, D), jnp.float32),
            pltpu.VMEM((BPW, D), jnp.float32),
            pltpu.VMEM((BPW,), jnp.int32),
            pltpu.VMEM((16,), jnp.float32),
            pltpu.SemaphoreType.DMA,
        ],
    )
    def _sc_gather(mapped_hbm, idx_hbm, ze_hbm, tok_hbm,
                   zq_out, tok_out, loss_out,
                   idx_v, rows_v, ze_v, tokout_v, acc_v, sem):
        wid = lax.axis_index("s") * 2 + lax.axis_index("c")
        base = wid * BPW
        pltpu.sync_copy(idx_hbm.at[pl.ds(base, BPW)], idx_v)
        pltpu.sync_copy(ze_hbm.at[pl.ds(base, BPW)], ze_v)
        pltpu.async_copy(mapped_hbm.at[idx_v], rows_v, sem).wait()
        pltpu.async_copy(tok_hbm.at[idx_v], tokout_v, sem).wait()
        acc_v[...] = jnp.zeros((16,), jnp.float32)

        def row_body(r, carry):
            for c in range(D // 16):
                sl = pl.ds(c * 16, 16)
                g = rows_v[r, sl]
                z = ze_v[r, sl]
                dlt = g - z
                ze_v[r, sl] = z + dlt
                acc_v[...] = acc_v[...] + dlt * dlt
            return carry

        lax.fori_loop(0, BPW, row_body, 0)

        pltpu.sync_copy(ze_v, zq_out.at[pl.ds(base, BPW)])
        pltpu.sync_copy(tokout_v, tok_out.at[pl.ds(base, BPW)])
        pltpu.sync_copy(acc_v, loss_out.at[wid])

    return _sc_gather


def kernel(z_e, codebook_pca, W, b, token_id):
    b2 = b.reshape(1, D)

    mapped, t8v, t8i = pl.pallas_call(
        _topk_body,
        grid=(K_TILES, N_TILES),
        in_specs=[
            pl.BlockSpec((KT, LLM), lambda k, j: (k, 0)),
            pl.BlockSpec((D, LLM), lambda k, j: (0, 0)),
            pl.BlockSpec((1, D), lambda k, j: (0, 0)),
            pl.BlockSpec((NB, D), lambda k, j: (j, 0)),
        ],
        out_specs=[
            pl.BlockSpec((KT, D), lambda k, j: (k, 0)),
            pl.BlockSpec((NB, WORD), lambda k, j: (j, 0)),
            pl.BlockSpec((NB, WORD), lambda k, j: (j, 0)),
        ],
        out_shape=[
            jax.ShapeDtypeStruct((K, D), jnp.float32),
            jax.ShapeDtypeStruct((N, WORD), jnp.float32),
            jax.ShapeDtypeStruct((N, WORD), jnp.int32),
        ],
        scratch_shapes=[
            pltpu.VMEM((KT, D), jnp.float32),
            pltpu.VMEM((1, KT), jnp.float32),
            pltpu.VMEM((N, WORD), jnp.float32),
            pltpu.VMEM((N, WORD), jnp.int32),
        ],
        compiler_params=pltpu.CompilerParams(
            dimension_semantics=("arbitrary", "arbitrary")),
    )(codebook_pca, W, b2, z_e)

    chosen = pl.pallas_call(
        _greedy_body,
        out_shape=jax.ShapeDtypeStruct((B_GROUPS, WORD), jnp.int32),
    )(t8v.reshape(B_GROUPS, WORD * WORD), t8i.reshape(B_GROUPS, WORD * WORD))

    idx = chosen.reshape(N)
    zq_st, tok_flat, partials = _build_sc_gather()(mapped, idx, z_e, token_id)

    l = jnp.sum(partials) / jnp.float32(N * D)
    loss = jnp.float32(0.75) * l + jnp.float32(0.25) * l
    return (zq_st, loss, tok_flat.reshape(N, 1), z_e[:, None, :])


# speedup vs baseline: 2.5152x; 1.1981x over previous
import functools

import jax
import jax.numpy as jnp
from jax import lax
from jax.experimental import pallas as pl
from jax.experimental.pallas import tpu as pltpu
from jax.experimental.pallas import tpu_sc as plsc

N, D, K, LLM = 4096, 256, 8192, 4096
WORD = 8
B_GROUPS = N // WORD
KT = 512
K_TILES = K // KT
BIG = 0x7FFFFFFF

NW = 32
BPW = N // NW


NB = 512
N_TILES = N // NB


def _topk_body(cb_ref, w_ref, b_ref, z_ref, mapped_ref, t8v_out, t8i_out,
               m_ref, msq_ref, t8v_ref, t8i_ref, vbuf_ref, gbuf_ref):
    k = pl.program_id(0)
    j = pl.program_id(1)

    @pl.when(k == 0)
    def _init():
        sl = pl.ds(j * NB, NB)
        t8v_ref[sl, :] = jnp.full((NB, WORD), jnp.inf, jnp.float32)
        t8i_ref[sl, :] = jnp.zeros((NB, WORD), jnp.int32)

    @pl.when(j == 0)
    def _mapped():
        m = lax.dot_general(cb_ref[...], w_ref[...], (((1,), (1,)), ((), ())),
                            preferred_element_type=jnp.float32) + b_ref[...]
        mapped_ref[...] = m
        m_ref[...] = m
        msq_ref[...] = jnp.sum(m * m, axis=1)[None, :]

    z = z_ref[...]
    zsq = jnp.sum(z * z, axis=1, keepdims=True)
    m = m_ref[...]
    dot = lax.dot_general(z, m, (((1,), (1,)), ((), ())),
                          preferred_element_type=jnp.float32)
    dist = (zsq + msq_ref[...]) - 2.0 * dot

    sl = pl.ds(j * NB, NB)
    vbuf_ref[:, :WORD] = t8v_ref[sl, :]
    gbuf_ref[:, :WORD] = t8i_ref[sl, :]
    vbuf_ref[:, WORD:] = dist
    gbuf_ref[:, WORD:] = k * KT + lax.broadcasted_iota(jnp.int32, (NB, KT), 1)

    for mx in range(WORD):
        @pl.when(j >= mx)
        def _extract(mx=mx):
            vals = vbuf_ref[...]
            glob = gbuf_ref[...]
            mn = jnp.min(vals, axis=1, keepdims=True)
            eq = vals == mn
            ch = jnp.min(jnp.where(eq, glob, BIG), axis=1, keepdims=True)
            t8v_ref[sl, mx:mx + 1] = mn
            t8i_ref[sl, mx:mx + 1] = ch
            vbuf_ref[...] = jnp.where(eq & (glob == ch), jnp.inf, vals)

    t8v_out[...] = t8v_ref[sl, :]
    t8i_out[...] = t8i_ref[sl, :]


def _greedy_body(v_ref, i_ref, out_ref):
    chosen = []
    for j in range(WORD):
        vj = v_ref[j * B_GROUPS:(j + 1) * B_GROUPS, :]
        ij = i_ref[j * B_GROUPS:(j + 1) * B_GROUPS, :]
        eff = vj
        for ch_prev in chosen:
            eff = jnp.where(ij == ch_prev, jnp.inf, eff)
        mn = jnp.min(eff, axis=1, keepdims=True)
        ch = jnp.min(jnp.where(eff == mn, ij, BIG), axis=1, keepdims=True)
        chosen.append(ch)
    out_ref[...] = jnp.concatenate(chosen, axis=1)


@functools.cache
def _build_sc_gather():
    mesh = plsc.VectorSubcoreMesh(core_axis_name="c", subcore_axis_name="s")

    @functools.partial(
        pl.kernel,
        mesh=mesh,
        out_type=[
            jax.ShapeDtypeStruct((N, D), jnp.float32),
            jax.ShapeDtypeStruct((N,), jnp.int32),
            jax.ShapeDtypeStruct((NW, 16), jnp.float32),
        ],
        scratch_types=[
            pltpu.VMEM((BPW,), jnp.int32),
            pltpu.VMEM((BPW, D), jnp.float32),
            pltpu.VMEM((BPW, D), jnp.float32),
            pltpu.VMEM((BPW,), jnp.int32),
            pltpu.VMEM((16,), jnp.float32),
            pltpu.SemaphoreType.DMA,
        ],
    )
    def _sc_gather(mapped_hbm, idx_hbm, ze_hbm, tok_hbm,
                   zq_out, tok_out, loss_out,
                   idx_v, rows_v, ze_v, tokout_v, acc_v, sem):
        wid = lax.axis_index("s") * 2 + lax.axis_index("c")
        base = wid * BPW
        pltpu.sync_copy(idx_hbm.at[pl.ds(base, BPW)], idx_v)
        pltpu.sync_copy(ze_hbm.at[pl.ds(base, BPW)], ze_v)
        pltpu.async_copy(mapped_hbm.at[idx_v], rows_v, sem).wait()
        pltpu.async_copy(tok_hbm.at[idx_v], tokout_v, sem).wait()
        acc_v[...] = jnp.zeros((16,), jnp.float32)

        def row_body(r, carry):
            for c in range(D // 16):
                sl = pl.ds(c * 16, 16)
                g = rows_v[r, sl]
                z = ze_v[r, sl]
                dlt = g - z
                ze_v[r, sl] = z + dlt
                acc_v[...] = acc_v[...] + dlt * dlt
            return carry

        lax.fori_loop(0, BPW, row_body, 0)

        pltpu.sync_copy(ze_v, zq_out.at[pl.ds(base, BPW)])
        pltpu.sync_copy(tokout_v, tok_out.at[pl.ds(base, BPW)])
        pltpu.sync_copy(acc_v, loss_out.at[wid])

    return _sc_gather


def kernel(z_e, codebook_pca, W, b, token_id):
    b2 = b.reshape(1, D)
    z_sm = z_e.reshape(B_GROUPS, WORD, D).transpose(1, 0, 2).reshape(N, D)

    mapped, t8v, t8i = pl.pallas_call(
        _topk_body,
        grid=(K_TILES, N_TILES),
        in_specs=[
            pl.BlockSpec((KT, LLM), lambda k, j: (k, 0)),
            pl.BlockSpec((D, LLM), lambda k, j: (0, 0)),
            pl.BlockSpec((1, D), lambda k, j: (0, 0)),
            pl.BlockSpec((NB, D), lambda k, j: (j, 0)),
        ],
        out_specs=[
            pl.BlockSpec((KT, D), lambda k, j: (k, 0)),
            pl.BlockSpec((NB, WORD), lambda k, j: (j, 0)),
            pl.BlockSpec((NB, WORD), lambda k, j: (j, 0)),
        ],
        out_shape=[
            jax.ShapeDtypeStruct((K, D), jnp.float32),
            jax.ShapeDtypeStruct((N, WORD), jnp.float32),
            jax.ShapeDtypeStruct((N, WORD), jnp.int32),
        ],
        scratch_shapes=[
            pltpu.VMEM((KT, D), jnp.float32),
            pltpu.VMEM((1, KT), jnp.float32),
            pltpu.VMEM((N, WORD), jnp.float32),
            pltpu.VMEM((N, WORD), jnp.int32),
            pltpu.VMEM((NB, KT + WORD), jnp.float32),
            pltpu.VMEM((NB, KT + WORD), jnp.int32),
        ],
        compiler_params=pltpu.CompilerParams(
            dimension_semantics=("arbitrary", "arbitrary")),
    )(codebook_pca, W, b2, z_sm)

    chosen = pl.pallas_call(
        _greedy_body,
        out_shape=jax.ShapeDtypeStruct((B_GROUPS, WORD), jnp.int32),
    )(t8v, t8i)

    idx = chosen.reshape(N)
    zq_st, tok_flat, partials = _build_sc_gather()(mapped, idx, z_e, token_id)

    l = jnp.sum(partials) / jnp.float32(N * D)
    loss = jnp.float32(0.75) * l + jnp.float32(0.25) * l
    return (zq_st, loss, tok_flat.reshape(N, 1), z_e[:, None, :])


# speedup vs baseline: 2.8072x; 1.1161x over previous
import functools

import jax
import jax.numpy as jnp
from jax import lax
from jax.experimental import pallas as pl
from jax.experimental.pallas import tpu as pltpu
from jax.experimental.pallas import tpu_sc as plsc

N, D, K, LLM = 4096, 256, 8192, 4096
WORD = 8
B_GROUPS = N // WORD
KT = 512
K_TILES = K // KT
BIG = 0x7FFFFFFF

NW = 32
BPW = N // NW


NB = 512
N_TILES = N // NB


def _topk_body(cb_ref, w_ref, b_ref, z_ref, mapped_ref, t8v_out, t8i_out,
               m_ref, msq_ref, t8v_ref, t8i_ref, vbuf_ref, gbuf_ref):
    k = pl.program_id(0)
    j = pl.program_id(1)

    @pl.when(k == 0)
    def _init():
        sl = pl.ds(j * NB, NB)
        t8v_ref[sl, :] = jnp.full((NB, WORD), jnp.inf, jnp.float32)
        t8i_ref[sl, :] = jnp.zeros((NB, WORD), jnp.float32)

    @pl.when(j == 0)
    def _mapped():
        m = lax.dot_general(cb_ref[...], w_ref[...], (((1,), (1,)), ((), ())),
                            preferred_element_type=jnp.float32) + b_ref[...]
        mapped_ref[...] = m
        m_ref[...] = m
        msq_ref[...] = jnp.sum(m * m, axis=1)[None, :]

    z = z_ref[...]
    zsq = jnp.sum(z * z, axis=1, keepdims=True)
    m = m_ref[...]
    dot = lax.dot_general(z, m, (((1,), (1,)), ((), ())),
                          preferred_element_type=jnp.float32)
    dist = (zsq + msq_ref[...]) - 2.0 * dot

    sl = pl.ds(j * NB, NB)
    vbuf_ref[:, :WORD] = t8v_ref[sl, :]
    gbuf_ref[:, :WORD] = t8i_ref[sl, :]
    vbuf_ref[:, WORD:] = dist
    gbuf_ref[:, WORD:] = jnp.float32(k * KT) + lax.broadcasted_iota(
        jnp.int32, (NB, KT), 1).astype(jnp.float32)

    FBIG = jnp.float32(1e9)
    for mx in range(WORD):
        @pl.when(j >= mx)
        def _extract(mx=mx):
            vals = vbuf_ref[...]
            glob = gbuf_ref[...]
            mn = jnp.min(vals, axis=1, keepdims=True)
            eq = vals == mn
            ch = jnp.min(jnp.where(eq, glob, FBIG), axis=1, keepdims=True)
            t8v_ref[sl, mx:mx + 1] = mn
            t8i_ref[sl, mx:mx + 1] = ch
            vbuf_ref[...] = jnp.where(eq & (glob == ch), jnp.inf, vals)

    t8v_out[...] = t8v_ref[sl, :]
    t8i_out[...] = t8i_ref[sl, :].astype(jnp.int32)


def _greedy_body(v_ref, i_ref, out_ref):
    chosen = []
    for j in range(WORD):
        vj = v_ref[j * B_GROUPS:(j + 1) * B_GROUPS, :]
        ij = i_ref[j * B_GROUPS:(j + 1) * B_GROUPS, :]
        eff = vj
        for ch_prev in chosen:
            eff = jnp.where(ij == ch_prev, jnp.inf, eff)
        mn = jnp.min(eff, axis=1, keepdims=True)
        ch = jnp.min(jnp.where(eff == mn, ij, BIG), axis=1, keepdims=True)
        chosen.append(ch)
    out_ref[...] = jnp.concatenate(chosen, axis=1)


@functools.cache
def _build_sc_gather():
    mesh = plsc.VectorSubcoreMesh(core_axis_name="c", subcore_axis_name="s")

    @functools.partial(
        pl.kernel,
        mesh=mesh,
        out_type=[
            jax.ShapeDtypeStruct((N, D), jnp.float32),
            jax.ShapeDtypeStruct((N,), jnp.int32),
            jax.ShapeDtypeStruct((NW, 16), jnp.float32),
        ],
        scratch_types=[
            pltpu.VMEM((BPW,), jnp.int32),
            pltpu.VMEM((BPW, D), jnp.float32),
            pltpu.VMEM((BPW, D), jnp.float32),
            pltpu.VMEM((BPW,), jnp.int32),
            pltpu.VMEM((16,), jnp.float32),
            pltpu.SemaphoreType.DMA,
        ],
    )
    def _sc_gather(mapped_hbm, idx_hbm, ze_hbm, tok_hbm,
                   zq_out, tok_out, loss_out,
                   idx_v, rows_v, ze_v, tokout_v, acc_v, sem):
        wid = lax.axis_index("s") * 2 + lax.axis_index("c")
        base = wid * BPW
        pltpu.sync_copy(idx_hbm.at[pl.ds(base, BPW)], idx_v)
        pltpu.sync_copy(ze_hbm.at[pl.ds(base, BPW)], ze_v)
        pltpu.async_copy(mapped_hbm.at[idx_v], rows_v, sem).wait()
        pltpu.async_copy(tok_hbm.at[idx_v], tokout_v, sem).wait()
        acc_v[...] = jnp.zeros((16,), jnp.float32)

        def row_body(r, carry):
            for c in range(D // 16):
                sl = pl.ds(c * 16, 16)
                g = rows_v[r, sl]
                z = ze_v[r, sl]
                dlt = g - z
                ze_v[r, sl] = z + dlt
                acc_v[...] = acc_v[...] + dlt * dlt
            return carry

        lax.fori_loop(0, BPW, row_body, 0)

        pltpu.sync_copy(ze_v, zq_out.at[pl.ds(base, BPW)])
        pltpu.sync_copy(tokout_v, tok_out.at[pl.ds(base, BPW)])
        pltpu.sync_copy(acc_v, loss_out.at[wid])

    return _sc_gather


def kernel(z_e, codebook_pca, W, b, token_id):
    b2 = b.reshape(1, D)
    z_sm = z_e.reshape(B_GROUPS, WORD, D).transpose(1, 0, 2).reshape(N, D)

    mapped, t8v, t8i = pl.pallas_call(
        _topk_body,
        grid=(K_TILES, N_TILES),
        in_specs=[
            pl.BlockSpec((KT, LLM), lambda k, j: (k, 0)),
            pl.BlockSpec((D, LLM), lambda k, j: (0, 0)),
            pl.BlockSpec((1, D), lambda k, j: (0, 0)),
            pl.BlockSpec((NB, D), lambda k, j: (j, 0)),
        ],
        out_specs=[
            pl.BlockSpec((KT, D), lambda k, j: (k, 0)),
            pl.BlockSpec((NB, WORD), lambda k, j: (j, 0)),
            pl.BlockSpec((NB, WORD), lambda k, j: (j, 0)),
        ],
        out_shape=[
            jax.ShapeDtypeStruct((K, D), jnp.float32),
            jax.ShapeDtypeStruct((N, WORD), jnp.float32),
            jax.ShapeDtypeStruct((N, WORD), jnp.int32),
        ],
        scratch_shapes=[
            pltpu.VMEM((KT, D), jnp.float32),
            pltpu.VMEM((1, KT), jnp.float32),
            pltpu.VMEM((N, WORD), jnp.float32),
            pltpu.VMEM((N, WORD), jnp.float32),
            pltpu.VMEM((NB, KT + WORD), jnp.float32),
            pltpu.VMEM((NB, KT + WORD), jnp.float32),
        ],
        compiler_params=pltpu.CompilerParams(
            dimension_semantics=("arbitrary", "arbitrary")),
    )(codebook_pca, W, b2, z_sm)

    chosen = pl.pallas_call(
        _greedy_body,
        out_shape=jax.ShapeDtypeStruct((B_GROUPS, WORD), jnp.int32),
    )(t8v, t8i)

    idx = chosen.reshape(N)
    zq_st, tok_flat, partials = _build_sc_gather()(mapped, idx, z_e, token_id)

    l = jnp.sum(partials) / jnp.float32(N * D)
    loss = jnp.float32(0.75) * l + jnp.float32(0.25) * l
    return (zq_st, loss, tok_flat.reshape(N, 1), z_e[:, None, :])


# speedup vs baseline: 3.2906x; 1.1722x over previous
import functools

import jax
import jax.numpy as jnp
from jax import lax
from jax.experimental import pallas as pl
from jax.experimental.pallas import tpu as pltpu
from jax.experimental.pallas import tpu_sc as plsc

N, D, K, LLM = 4096, 256, 8192, 4096
WORD = 8
B_GROUPS = N // WORD
KT = 1024
K_TILES = K // KT
BIG = 0x7FFFFFFF

NW = 32
BPW = N // NW


NB = 512
N_TILES = N // NB


def _topk_body(cb_ref, w_ref, b_ref, z_ref, mapped_ref, t8v_out, t8i_out,
               m_ref, msq_ref, t8v_ref, t8i_ref, vbuf_ref, gbuf_ref):
    k = pl.program_id(0)
    j = pl.program_id(1)

    @pl.when(k == 0)
    def _init():
        sl = pl.ds(j * NB, NB)
        t8v_ref[sl, :] = jnp.full((NB, WORD), jnp.inf, jnp.float32)
        t8i_ref[sl, :] = jnp.zeros((NB, WORD), jnp.float32)

    @pl.when(j == 0)
    def _mapped():
        m = lax.dot_general(cb_ref[...], w_ref[...], (((1,), (1,)), ((), ())),
                            preferred_element_type=jnp.float32) + b_ref[...]
        mapped_ref[...] = m
        m_ref[...] = m
        msq_ref[...] = jnp.sum(m * m, axis=1)[None, :]

    z = z_ref[...]
    zsq = jnp.sum(z * z, axis=1, keepdims=True)
    m = m_ref[...]
    dot = lax.dot_general(z, m, (((1,), (1,)), ((), ())),
                          preferred_element_type=jnp.float32)
    dist = (zsq + msq_ref[...]) - 2.0 * dot

    sl = pl.ds(j * NB, NB)
    vbuf_ref[:, :WORD] = t8v_ref[sl, :]
    gbuf_ref[:, :WORD] = t8i_ref[sl, :]
    vbuf_ref[:, WORD:] = dist
    gbuf_ref[:, WORD:] = jnp.float32(k * KT) + lax.broadcasted_iota(
        jnp.int32, (NB, KT), 1).astype(jnp.float32)

    FBIG = jnp.float32(1e9)
    for mx in range(WORD):
        @pl.when(j >= mx)
        def _extract(mx=mx):
            vals = vbuf_ref[...]
            glob = gbuf_ref[...]
            mn = jnp.min(vals, axis=1, keepdims=True)
            eq = vals == mn
            ch = jnp.min(jnp.where(eq, glob, FBIG), axis=1, keepdims=True)
            t8v_ref[sl, mx:mx + 1] = mn
            t8i_ref[sl, mx:mx + 1] = ch
            vbuf_ref[...] = jnp.where(eq & (glob == ch), jnp.inf, vals)

    t8v_out[...] = t8v_ref[sl, :]
    t8i_out[...] = t8i_ref[sl, :].astype(jnp.int32)


def _greedy_body(v_ref, i_ref, out_ref):
    chosen = []
    for j in range(WORD):
        vj = v_ref[j * B_GROUPS:(j + 1) * B_GROUPS, :]
        ij = i_ref[j * B_GROUPS:(j + 1) * B_GROUPS, :]
        eff = vj
        for ch_prev in chosen:
            eff = jnp.where(ij == ch_prev, jnp.inf, eff)
        mn = jnp.min(eff, axis=1, keepdims=True)
        ch = jnp.min(jnp.where(eff == mn, ij, BIG), axis=1, keepdims=True)
        chosen.append(ch)
    out_ref[...] = jnp.concatenate(chosen, axis=1)


@functools.cache
def _build_sc_gather():
    mesh = plsc.VectorSubcoreMesh(core_axis_name="c", subcore_axis_name="s")

    @functools.partial(
        pl.kernel,
        mesh=mesh,
        out_type=[
            jax.ShapeDtypeStruct((N, D), jnp.float32),
            jax.ShapeDtypeStruct((N,), jnp.int32),
            jax.ShapeDtypeStruct((NW, 16), jnp.float32),
        ],
        scratch_types=[
            pltpu.VMEM((BPW,), jnp.int32),
            pltpu.VMEM((BPW, D), jnp.float32),
            pltpu.VMEM((BPW, D), jnp.float32),
            pltpu.VMEM((BPW,), jnp.int32),
            pltpu.VMEM((16,), jnp.float32),
            pltpu.SemaphoreType.DMA,
        ],
    )
    def _sc_gather(mapped_hbm, idx_hbm, ze_hbm, tok_hbm,
                   zq_out, tok_out, loss_out,
                   idx_v, rows_v, ze_v, tokout_v, acc_v, sem):
        wid = lax.axis_index("s") * 2 + lax.axis_index("c")
        base = wid * BPW
        pltpu.sync_copy(idx_hbm.at[pl.ds(base, BPW)], idx_v)
        pltpu.sync_copy(ze_hbm.at[pl.ds(base, BPW)], ze_v)
        pltpu.async_copy(mapped_hbm.at[idx_v], rows_v, sem).wait()
        pltpu.async_copy(tok_hbm.at[idx_v], tokout_v, sem).wait()
        acc_v[...] = jnp.zeros((16,), jnp.float32)

        def row_body(r, carry):
            for c in range(D // 16):
                sl = pl.ds(c * 16, 16)
                g = rows_v[r, sl]
                z = ze_v[r, sl]
                dlt = g - z
                ze_v[r, sl] = z + dlt
                acc_v[...] = acc_v[...] + dlt * dlt
            return carry

        lax.fori_loop(0, BPW, row_body, 0)

        pltpu.sync_copy(ze_v, zq_out.at[pl.ds(base, BPW)])
        pltpu.sync_copy(tokout_v, tok_out.at[pl.ds(base, BPW)])
        pltpu.sync_copy(acc_v, loss_out.at[wid])

    return _sc_gather


def kernel(z_e, codebook_pca, W, b, token_id):
    b2 = b.reshape(1, D)
    z_sm = z_e.reshape(B_GROUPS, WORD, D).transpose(1, 0, 2).reshape(N, D)

    mapped, t8v, t8i = pl.pallas_call(
        _topk_body,
        grid=(K_TILES, N_TILES),
        in_specs=[
            pl.BlockSpec((KT, LLM), lambda k, j: (k, 0)),
            pl.BlockSpec((D, LLM), lambda k, j: (0, 0)),
            pl.BlockSpec((1, D), lambda k, j: (0, 0)),
            pl.BlockSpec((NB, D), lambda k, j: (j, 0)),
        ],
        out_specs=[
            pl.BlockSpec((KT, D), lambda k, j: (k, 0)),
            pl.BlockSpec((NB, WORD), lambda k, j: (j, 0)),
            pl.BlockSpec((NB, WORD), lambda k, j: (j, 0)),
        ],
        out_shape=[
            jax.ShapeDtypeStruct((K, D), jnp.float32),
            jax.ShapeDtypeStruct((N, WORD), jnp.float32),
            jax.ShapeDtypeStruct((N, WORD), jnp.int32),
        ],
        scratch_shapes=[
            pltpu.VMEM((KT, D), jnp.float32),
            pltpu.VMEM((1, KT), jnp.float32),
            pltpu.VMEM((N, WORD), jnp.float32),
            pltpu.VMEM((N, WORD), jnp.float32),
            pltpu.VMEM((NB, KT + WORD), jnp.float32),
            pltpu.VMEM((NB, KT + WORD), jnp.float32),
        ],
        compiler_params=pltpu.CompilerParams(
            dimension_semantics=("arbitrary", "arbitrary")),
    )(codebook_pca, W, b2, z_sm)

    chosen = pl.pallas_call(
        _greedy_body,
        out_shape=jax.ShapeDtypeStruct((B_GROUPS, WORD), jnp.int32),
    )(t8v, t8i)

    idx = chosen.reshape(N)
    zq_st, tok_flat, partials = _build_sc_gather()(mapped, idx, z_e, token_id)

    l = jnp.sum(partials) / jnp.float32(N * D)
    loss = jnp.float32(0.75) * l + jnp.float32(0.25) * l
    return (zq_st, loss, tok_flat.reshape(N, 1), z_e[:, None, :])
